# baseline (speedup 1.0000x reference)
import jax
import jax.numpy as jnp
from jax.experimental import pallas as pl
from jax.experimental.pallas import tpu as pltpu

_WB = 64


def _paired_matmuls(pae_ref, pao_ref, pbe_ref, pbo_ref, wa_ref, wb_ref,
                    K, M):
    acc = None
    for j in range(K + 1):
        pa = pae_ref if j % 2 == 0 else pao_ref
        pb = pbe_ref if j % 2 == 0 else pbo_ref
        s = (j // 2) * _WB
        t = (jnp.dot(pa[s:s + M, :], wa_ref[j],
                     preferred_element_type=jnp.float32) +
             jnp.dot(pb[s:s + M, :], wb_ref[j],
                     preferred_element_type=jnp.float32))
        acc = t if acc is None else acc + t
    return acc


def _conv1_kernel(H, W, C, K):
    Hp = H + 2 * (K // 2)
    NA, NB = (K + 1) // 2, K // 2
    HPE = (Hp // 2) * _WB
    M = (H // 2) * _WB

    def body(x_ref, wa_ref, wb_ref, b_ref, y_ref, st_ref,
             x1_ref, pae_ref, pao_ref, pbe_ref, pbo_ref):
        nr = Hp * _WB
        for i in range(x_ref.shape[0]):
            x1_ref[0:x1_ref.shape[0] - 1, :] = x_ref[i, 1:, :]
            for a in range(NA):
                seg = x_ref[i, 2 * a:2 * a + nr, :].reshape(
                    Hp // 2, 2, _WB, C)
                pae_ref[:, a * C:(a + 1) * C] = seg[:, 0].reshape(HPE, C)
                pao_ref[:, a * C:(a + 1) * C] = seg[:, 1].reshape(HPE, C)
            for b in range(NB):
                seg = x1_ref[2 * b:2 * b + nr, :].reshape(
                    Hp // 2, 2, _WB, C)
                pbe_ref[:, b * C:(b + 1) * C] = seg[:, 0].reshape(HPE, C)
                pbo_ref[:, b * C:(b + 1) * C] = seg[:, 1].reshape(HPE, C)
            acc = _paired_matmuls(pae_ref, pao_ref, pbe_ref, pbo_ref,
                                  wa_ref, wb_ref, K, M)
            acc = jnp.maximum(acc + b_ref[...], 0.0)
            a3 = acc.reshape(H // 2, _WB, 2 * C)
            msk = jax.lax.broadcasted_iota(jnp.int32, a3.shape, 1) < W
            acc = jnp.where(msk, a3, 0.0).reshape(M, 2 * C)
            y_ref[i] = acc.astype(jnp.bfloat16)
            st_ref[i] = jnp.concatenate(
                [jnp.sum(acc, axis=0, keepdims=True),
                 jnp.sum(acc * acc, axis=0, keepdims=True)], axis=0)

    return body


def _conv2_pool_kernel(H, W, C, K):
    PAD = K // 2
    Hp = H + 2 * PAD
    Ho, Wo = H // 2, W // 2
    M = Ho * _WB
    HPE = (Hp // 2) * _WB
    NA, NB = K // 2, (K + 1) // 2

    def body(y1_ref, corr_ref, wa_ref, wb_ref, y2_ref, st_ref,
             xpe_ref, xpo_ref, xpe1_ref, xpo1_ref,
             pae_ref, pao_ref, pbe_ref, pbo_ref):
        @pl.when(pl.program_id(0) == 0)
        def _zero_guards():
            xpe_ref[...] = jnp.zeros_like(xpe_ref)
            xpo_ref[...] = jnp.zeros_like(xpo_ref)
            xpe1_ref[...] = jnp.zeros_like(xpe1_ref)
            xpo1_ref[...] = jnp.zeros_like(xpo1_ref)
        eoff = 2 * _WB + PAD + 1
        ooff = _WB + PAD + 1
        for i in range(y1_ref.shape[0]):
            zb = y1_ref[i]
            zl, zr = zb[:, :C], zb[:, C:]
            xpe_ref[eoff:eoff + M, :] = zr
            xpe1_ref[eoff - 1:eoff - 1 + M, :] = zr
            xpo_ref[ooff:ooff + M, :] = zl
            xpo1_ref[ooff - 1:ooff - 1 + M, :] = zl
            for a in range(NA):
                pae_ref[:, a * C:(a + 1) * C] = xpe_ref[2 * a + 2:
                                                        2 * a + 2 + HPE, :]
                pao_ref[:, a * C:(a + 1) * C] = xpo_ref[2 * a + 2:
                                                        2 * a + 2 + HPE, :]
            for b in range(NB):
                pbe_ref[:, b * C:(b + 1) * C] = xpe1_ref[2 * b:
                                                         2 * b + HPE, :]
                pbo_ref[:, b * C:(b + 1) * C] = xpo1_ref[2 * b:
                                                         2 * b + HPE, :]
            acc = _paired_matmuls(pae_ref, pao_ref, pbe_ref, pbo_ref,
                                  wa_ref, wb_ref, K, M)
            acc = jnp.maximum(acc + corr_ref[...], 0.0)
            ph = jnp.maximum(acc[:, :C], acc[:, C:])
            pw = ph.reshape(Ho, _WB // 2, 2, C)
            pooled = jnp.maximum(pw[:, :, 0], pw[:, :, 1])
            pmsk = jax.lax.broadcasted_iota(jnp.int32, pooled.shape, 1) < Wo
            pooled = jnp.where(pmsk, pooled, 0.0).reshape(
                Ho * (_WB // 2), C)
            y2_ref[i] = pooled.astype(jnp.bfloat16)
            st_ref[i] = jnp.concatenate(
                [jnp.sum(pooled, axis=0, keepdims=True),
                 jnp.sum(pooled * pooled, axis=0, keepdims=True)], axis=0)

    return body


def _affine_kernel(y_ref, sc_ref, sh_ref, o_ref):
    o_ref[...] = y_ref[...].astype(jnp.float32) * sc_ref[...] + sh_ref[...]


def _pair_weights(wr):
    z = jnp.zeros_like(wr[:1])
    left = jnp.concatenate([wr, z], axis=0)
    right = jnp.concatenate([z, wr], axis=0)
    return jnp.concatenate([left, right], axis=2).astype(jnp.bfloat16)


def _bn_scale_shift(stats, count, gamma, beta, C, eps=1e-5):
    s = jnp.sum(stats, axis=0)
    if s.shape[-1] == 2 * C:
        s = s[:, :C] + s[:, C:]
    mean = s[0] / count
    var = jnp.maximum(s[1] / count - mean * mean, 0.0)
    scale = gamma * jax.lax.rsqrt(var + eps)
    shift = beta - mean * scale
    return (scale.reshape(1, -1).astype(jnp.float32),
            shift.reshape(1, -1).astype(jnp.float32))


def kernel(x, w1, b1, gamma1, beta1, w2, b2, gamma2, beta2):
    N, Cin, H, W = x.shape
    K = w1.shape[0]
    C = w1.shape[-1]
    PAD = K // 2
    Hp = H + 2 * PAD
    Ho, Wo = H // 2, W // 2
    NR = Hp * _WB + 8
    HPE = (Hp // 2) * _WB
    HPG = HPE + 16
    M = Ho * _WB

    xh = jnp.transpose(x, (0, 2, 3, 1)).astype(jnp.bfloat16)
    xp = jnp.pad(xh, ((0, 0), (PAD, PAD), (PAD, _WB - W - PAD), (0, 0)))
    xf = jnp.pad(xp.reshape(N, Hp * _WB, C), ((0, 0), (0, 8), (0, 0)))

    w1r = w1.astype(jnp.float32)
    w2r = w2.astype(jnp.float32)
    wa1 = _pair_weights(w1r[:, 0::2].reshape(K, -1, C))
    wb1 = _pair_weights(w1r[:, 1::2].reshape(K, -1, C))
    b1p = jnp.tile(b1.reshape(1, C), (1, 2)).astype(jnp.float32)
    NA1, NB1 = (K + 1) // 2, K // 2
    NA2, NB2 = K // 2, (K + 1) // 2

    parallel = pltpu.CompilerParams(dimension_semantics=("parallel",))
    IB = 4 if N % 4 == 0 else (2 if N % 2 == 0 else 1)

    y1, st1 = pl.pallas_call(
        _conv1_kernel(H, W, C, K),
        out_shape=(jax.ShapeDtypeStruct((N, M, 2 * C), jnp.bfloat16),
                   jax.ShapeDtypeStruct((N, 2, 2 * C), jnp.float32)),
        grid=(N // IB,),
        in_specs=[pl.BlockSpec((IB, NR, C), lambda n: (n, 0, 0)),
                  pl.BlockSpec((K + 1, NA1 * Cin, 2 * C),
                               lambda n: (0, 0, 0)),
                  pl.BlockSpec((K + 1, NB1 * Cin, 2 * C),
                               lambda n: (0, 0, 0)),
                  pl.BlockSpec((1, 2 * C), lambda n: (0, 0))],
        out_specs=(pl.BlockSpec((IB, M, 2 * C), lambda n: (n, 0, 0)),
                   pl.BlockSpec((IB, 2, 2 * C), lambda n: (n, 0, 0))),
        scratch_shapes=[pltpu.VMEM((NR, C), jnp.bfloat16),
                        pltpu.VMEM((HPE, NA1 * Cin), jnp.bfloat16),
                        pltpu.VMEM((HPE, NA1 * Cin), jnp.bfloat16),
                        pltpu.VMEM((HPE, NB1 * Cin), jnp.bfloat16),
                        pltpu.VMEM((HPE, NB1 * Cin), jnp.bfloat16)],
        compiler_params=parallel,
    )(xf, wa1, wb1, b1p)
    sc1, sh1 = _bn_scale_shift(st1, N * H * W, gamma1, beta1, C)

    w2s = w2r * sc1.reshape(1, 1, C, 1)
    wa2 = _pair_weights(w2s[:, 1::2].reshape(K, -1, C))
    wb2 = _pair_weights(w2s[:, 0::2].reshape(K, -1, C))
    m2 = jnp.einsum("klcd,c->kld", w2r, sh1.reshape(C))
    posh = jnp.arange(H).reshape(H, 1) + jnp.arange(K).reshape(1, K) - PAD
    vh = ((posh >= 0) & (posh < H)).astype(jnp.float32)
    posw = jnp.arange(W).reshape(W, 1) + jnp.arange(K).reshape(1, K) - PAD
    vw = ((posw >= 0) & (posw < W)).astype(jnp.float32)
    corr = jnp.einsum("hk,wl,kld->hwd", vh, vw, m2) + b2.reshape(1, 1, C)
    corr = jnp.pad(corr, ((0, 0), (0, _WB - W), (0, 0)))
    corr = corr.reshape(Ho, 2, _WB, C).transpose(0, 2, 1, 3).reshape(
        M, 2 * C).astype(jnp.float32)

    y2, st2 = pl.pallas_call(
        _conv2_pool_kernel(H, W, C, K),
        out_shape=(jax.ShapeDtypeStruct((N, Ho * (_WB // 2), C),
                                        jnp.bfloat16),
                   jax.ShapeDtypeStruct((N, 2, C), jnp.float32)),
        grid=(N // IB,),
        in_specs=[pl.BlockSpec((IB, M, 2 * C), lambda n: (n, 0, 0)),
                  pl.BlockSpec((M, 2 * C), lambda n: (0, 0)),
                  pl.BlockSpec((K + 1, NA2 * C, 2 * C), lambda n: (0, 0, 0)),
                  pl.BlockSpec((K + 1, NB2 * C, 2 * C), lambda n: (0, 0, 0))],
        out_specs=(pl.BlockSpec((IB, Ho * (_WB // 2), C),
                                lambda n: (n, 0, 0)),
                   pl.BlockSpec((IB, 2, C), lambda n: (n, 0, 0))),
        scratch_shapes=[pltpu.VMEM((HPG, C), jnp.bfloat16),
                        pltpu.VMEM((HPG, C), jnp.bfloat16),
                        pltpu.VMEM((HPG, C), jnp.bfloat16),
                        pltpu.VMEM((HPG, C), jnp.bfloat16),
                        pltpu.VMEM((HPE, NA2 * C), jnp.bfloat16),
                        pltpu.VMEM((HPE, NA2 * C), jnp.bfloat16),
                        pltpu.VMEM((HPE, NB2 * C), jnp.bfloat16),
                        pltpu.VMEM((HPE, NB2 * C), jnp.bfloat16)],
        compiler_params=parallel,
    )(y1, corr, wa2, wb2)
    sc2, sh2 = _bn_scale_shift(st2, N * Ho * Wo, gamma2, beta2, C)

    NBLK = 8 if N % 8 == 0 else 1
    WoB = _WB // 2
    out = pl.pallas_call(
        _affine_kernel,
        out_shape=jax.ShapeDtypeStruct((N, Ho * WoB, C), jnp.float32),
        grid=(N // NBLK,),
        in_specs=[pl.BlockSpec((NBLK, Ho * WoB, C), lambda n: (n, 0, 0)),
                  pl.BlockSpec((1, C), lambda n: (0, 0)),
                  pl.BlockSpec((1, C), lambda n: (0, 0))],
        out_specs=pl.BlockSpec((NBLK, Ho * WoB, C), lambda n: (n, 0, 0)),
        compiler_params=parallel,
    )(y2, sc2, sh2)

    out = out.reshape(N, Ho, WoB, C)[:, :, :Wo, :]
    return jnp.transpose(out, (0, 3, 1, 2))


# speedup vs baseline: 1.0027x; 1.0027x over previous
import jax
import jax.numpy as jnp
from jax.experimental import pallas as pl
from jax.experimental.pallas import tpu as pltpu

_WB = 64


def _paired_matmuls(pae_ref, pao_ref, pbe_ref, pbo_ref, wa_ref, wb_ref,
                    K, M):
    acc = None
    for j in range(K + 1):
        pa = pae_ref if j % 2 == 0 else pao_ref
        pb = pbe_ref if j % 2 == 0 else pbo_ref
        s = (j // 2) * _WB
        t = (jnp.dot(pa[s:s + M, :], wa_ref[j],
                     preferred_element_type=jnp.float32) +
             jnp.dot(pb[s:s + M, :], wb_ref[j],
                     preferred_element_type=jnp.float32))
        acc = t if acc is None else acc + t
    return acc


def _conv1_kernel(H, W, C, K):
    Hp = H + 2 * (K // 2)
    NA, NB = (K + 1) // 2, K // 2
    HPE = (Hp // 2) * _WB
    M = (H // 2) * _WB

    def body(x_ref, wa_ref, wb_ref, b_ref, y_ref, st_ref,
             x1_ref, pae_ref, pao_ref, pbe_ref, pbo_ref):
        nr = Hp * _WB
        for i in range(x_ref.shape[0]):
            x1_ref[0:x1_ref.shape[0] - 1, :] = x_ref[i, 1:, :]
            for a in range(NA):
                seg = x_ref[i, 2 * a:2 * a + nr, :].reshape(
                    Hp // 2, 2, _WB, C)
                pae_ref[:, a * C:(a + 1) * C] = seg[:, 0].reshape(HPE, C)
                pao_ref[:, a * C:(a + 1) * C] = seg[:, 1].reshape(HPE, C)
            for b in range(NB):
                seg = x1_ref[2 * b:2 * b + nr, :].reshape(
                    Hp // 2, 2, _WB, C)
                pbe_ref[:, b * C:(b + 1) * C] = seg[:, 0].reshape(HPE, C)
                pbo_ref[:, b * C:(b + 1) * C] = seg[:, 1].reshape(HPE, C)
            acc = _paired_matmuls(pae_ref, pao_ref, pbe_ref, pbo_ref,
                                  wa_ref, wb_ref, K, M)
            acc = jnp.maximum(acc + b_ref[...], 0.0)
            a3 = acc.reshape(H // 2, _WB, 2 * C)
            msk = jax.lax.broadcasted_iota(jnp.int32, a3.shape, 1) < W
            acc = jnp.where(msk, a3, 0.0).reshape(M, 2 * C)
            y_ref[i] = acc.astype(jnp.bfloat16)
            st_ref[i] = jnp.concatenate(
                [jnp.sum(acc, axis=0, keepdims=True),
                 jnp.sum(acc * acc, axis=0, keepdims=True)], axis=0)

    return body


def _conv2_pool_kernel(H, W, C, K):
    PAD = K // 2
    Hp = H + 2 * PAD
    Ho, Wo = H // 2, W // 2
    M = Ho * _WB
    HPE = (Hp // 2) * _WB
    NA, NB = K // 2, (K + 1) // 2

    def body(y1_ref, corr_ref, wa_ref, wb_ref, y2_ref, st_ref,
             xpe_ref, xpo_ref, xpe1_ref, xpo1_ref,
             pae_ref, pao_ref, pbe_ref, pbo_ref):
        @pl.when(pl.program_id(0) == 0)
        def _zero_guards():
            xpe_ref[...] = jnp.zeros_like(xpe_ref)
            xpo_ref[...] = jnp.zeros_like(xpo_ref)
            xpe1_ref[...] = jnp.zeros_like(xpe1_ref)
            xpo1_ref[...] = jnp.zeros_like(xpo1_ref)
        eoff = 2 * _WB + PAD + 1
        ooff = _WB + PAD + 1
        for i in range(y1_ref.shape[0]):
            zb = y1_ref[i]
            zl, zr = zb[:, :C], zb[:, C:]
            xpe_ref[eoff:eoff + M, :] = zr
            xpe1_ref[eoff - 1:eoff - 1 + M, :] = zr
            xpo_ref[ooff:ooff + M, :] = zl
            xpo1_ref[ooff - 1:ooff - 1 + M, :] = zl
            for a in range(NA):
                pae_ref[:, a * C:(a + 1) * C] = xpe_ref[2 * a + 2:
                                                        2 * a + 2 + HPE, :]
                pao_ref[:, a * C:(a + 1) * C] = xpo_ref[2 * a + 2:
                                                        2 * a + 2 + HPE, :]
            for b in range(NB):
                pbe_ref[:, b * C:(b + 1) * C] = xpe1_ref[2 * b:
                                                         2 * b + HPE, :]
                pbo_ref[:, b * C:(b + 1) * C] = xpo1_ref[2 * b:
                                                         2 * b + HPE, :]
            acc = _paired_matmuls(pae_ref, pao_ref, pbe_ref, pbo_ref,
                                  wa_ref, wb_ref, K, M)
            acc = jnp.maximum(acc + corr_ref[...], 0.0)
            ph = jnp.maximum(acc[:, :C], acc[:, C:])
            pw = ph.reshape(Ho, _WB // 2, 2, C)
            pooled = jnp.maximum(pw[:, :, 0], pw[:, :, 1])
            pmsk = jax.lax.broadcasted_iota(jnp.int32, pooled.shape, 1) < Wo
            pooled = jnp.where(pmsk, pooled, 0.0).reshape(
                Ho * (_WB // 2), C)
            y2_ref[i] = pooled.astype(jnp.bfloat16)
            st_ref[i] = jnp.concatenate(
                [jnp.sum(pooled, axis=0, keepdims=True),
                 jnp.sum(pooled * pooled, axis=0, keepdims=True)], axis=0)

    return body


def _affine_kernel(y_ref, sc_ref, sh_ref, o_ref):
    o_ref[...] = y_ref[...].astype(jnp.float32) * sc_ref[...] + sh_ref[...]


def _pair_weights(wr):
    z = jnp.zeros_like(wr[:1])
    left = jnp.concatenate([wr, z], axis=0)
    right = jnp.concatenate([z, wr], axis=0)
    return jnp.concatenate([left, right], axis=2).astype(jnp.bfloat16)


def _bn_scale_shift(stats, count, gamma, beta, C, eps=1e-5):
    s = jnp.sum(stats, axis=0)
    if s.shape[-1] == 2 * C:
        s = s[:, :C] + s[:, C:]
    mean = s[0] / count
    var = jnp.maximum(s[1] / count - mean * mean, 0.0)
    scale = gamma * jax.lax.rsqrt(var + eps)
    shift = beta - mean * scale
    return (scale.reshape(1, -1).astype(jnp.float32),
            shift.reshape(1, -1).astype(jnp.float32))


def kernel(x, w1, b1, gamma1, beta1, w2, b2, gamma2, beta2):
    N, Cin, H, W = x.shape
    K = w1.shape[0]
    C = w1.shape[-1]
    PAD = K // 2
    Hp = H + 2 * PAD
    Ho, Wo = H // 2, W // 2
    NR = Hp * _WB + 8
    HPE = (Hp // 2) * _WB
    HPG = HPE + 16
    M = Ho * _WB

    xh = jnp.transpose(x, (0, 2, 3, 1)).astype(jnp.bfloat16)
    xp = jnp.pad(xh, ((0, 0), (PAD, PAD), (PAD, _WB - W - PAD), (0, 0)))
    xf = jnp.pad(xp.reshape(N, Hp * _WB, C), ((0, 0), (0, 8), (0, 0)))

    w1r = w1.astype(jnp.float32)
    w2r = w2.astype(jnp.float32)
    wa1 = _pair_weights(w1r[:, 0::2].reshape(K, -1, C))
    wb1 = _pair_weights(w1r[:, 1::2].reshape(K, -1, C))
    b1p = jnp.tile(b1.reshape(1, C), (1, 2)).astype(jnp.float32)
    NA1, NB1 = (K + 1) // 2, K // 2
    NA2, NB2 = K // 2, (K + 1) // 2

    parallel = pltpu.CompilerParams(dimension_semantics=("parallel",))
    IB = 2 if N % 2 == 0 else 1

    y1, st1 = pl.pallas_call(
        _conv1_kernel(H, W, C, K),
        out_shape=(jax.ShapeDtypeStruct((N, M, 2 * C), jnp.bfloat16),
                   jax.ShapeDtypeStruct((N, 2, 2 * C), jnp.float32)),
        grid=(N // IB,),
        in_specs=[pl.BlockSpec((IB, NR, C), lambda n: (n, 0, 0)),
                  pl.BlockSpec((K + 1, NA1 * Cin, 2 * C),
                               lambda n: (0, 0, 0)),
                  pl.BlockSpec((K + 1, NB1 * Cin, 2 * C),
                               lambda n: (0, 0, 0)),
                  pl.BlockSpec((1, 2 * C), lambda n: (0, 0))],
        out_specs=(pl.BlockSpec((IB, M, 2 * C), lambda n: (n, 0, 0)),
                   pl.BlockSpec((IB, 2, 2 * C), lambda n: (n, 0, 0))),
        scratch_shapes=[pltpu.VMEM((NR, C), jnp.bfloat16),
                        pltpu.VMEM((HPE, NA1 * Cin), jnp.bfloat16),
                        pltpu.VMEM((HPE, NA1 * Cin), jnp.bfloat16),
                        pltpu.VMEM((HPE, NB1 * Cin), jnp.bfloat16),
                        pltpu.VMEM((HPE, NB1 * Cin), jnp.bfloat16)],
        compiler_params=parallel,
    )(xf, wa1, wb1, b1p)
    sc1, sh1 = _bn_scale_shift(st1, N * H * W, gamma1, beta1, C)

    w2s = w2r * sc1.reshape(1, 1, C, 1)
    wa2 = _pair_weights(w2s[:, 1::2].reshape(K, -1, C))
    wb2 = _pair_weights(w2s[:, 0::2].reshape(K, -1, C))
    m2 = jnp.einsum("klcd,c->kld", w2r, sh1.reshape(C))
    posh = jnp.arange(H).reshape(H, 1) + jnp.arange(K).reshape(1, K) - PAD
    vh = ((posh >= 0) & (posh < H)).astype(jnp.float32)
    posw = jnp.arange(W).reshape(W, 1) + jnp.arange(K).reshape(1, K) - PAD
    vw = ((posw >= 0) & (posw < W)).astype(jnp.float32)
    corr = jnp.einsum("hk,wl,kld->hwd", vh, vw, m2) + b2.reshape(1, 1, C)
    corr = jnp.pad(corr, ((0, 0), (0, _WB - W), (0, 0)))
    corr = corr.reshape(Ho, 2, _WB, C).transpose(0, 2, 1, 3).reshape(
        M, 2 * C).astype(jnp.float32)

    y2, st2 = pl.pallas_call(
        _conv2_pool_kernel(H, W, C, K),
        out_shape=(jax.ShapeDtypeStruct((N, Ho * (_WB // 2), C),
                                        jnp.bfloat16),
                   jax.ShapeDtypeStruct((N, 2, C), jnp.float32)),
        grid=(N // IB,),
        in_specs=[pl.BlockSpec((IB, M, 2 * C), lambda n: (n, 0, 0)),
                  pl.BlockSpec((M, 2 * C), lambda n: (0, 0)),
                  pl.BlockSpec((K + 1, NA2 * C, 2 * C), lambda n: (0, 0, 0)),
                  pl.BlockSpec((K + 1, NB2 * C, 2 * C), lambda n: (0, 0, 0))],
        out_specs=(pl.BlockSpec((IB, Ho * (_WB // 2), C),
                                lambda n: (n, 0, 0)),
                   pl.BlockSpec((IB, 2, C), lambda n: (n, 0, 0))),
        scratch_shapes=[pltpu.VMEM((HPG, C), jnp.bfloat16),
                        pltpu.VMEM((HPG, C), jnp.bfloat16),
                        pltpu.VMEM((HPG, C), jnp.bfloat16),
                        pltpu.VMEM((HPG, C), jnp.bfloat16),
                        pltpu.VMEM((HPE, NA2 * C), jnp.bfloat16),
                        pltpu.VMEM((HPE, NA2 * C), jnp.bfloat16),
                        pltpu.VMEM((HPE, NB2 * C), jnp.bfloat16),
                        pltpu.VMEM((HPE, NB2 * C), jnp.bfloat16)],
        compiler_params=parallel,
    )(y1, corr, wa2, wb2)
    sc2, sh2 = _bn_scale_shift(st2, N * Ho * Wo, gamma2, beta2, C)

    NBLK = 8 if N % 8 == 0 else 1
    WoB = _WB // 2
    out = pl.pallas_call(
        _affine_kernel,
        out_shape=jax.ShapeDtypeStruct((N, Ho * WoB, C), jnp.float32),
        grid=(N // NBLK,),
        in_specs=[pl.BlockSpec((NBLK, Ho * WoB, C), lambda n: (n, 0, 0)),
                  pl.BlockSpec((1, C), lambda n: (0, 0)),
                  pl.BlockSpec((1, C), lambda n: (0, 0))],
        out_specs=pl.BlockSpec((NBLK, Ho * WoB, C), lambda n: (n, 0, 0)),
        compiler_params=parallel,
    )(y2, sc2, sh2)

    out = out.reshape(N, Ho, WoB, C)[:, :, :Wo, :]
    return jnp.transpose(out, (0, 3, 1, 2))
